# baseline (speedup 1.0000x reference)
import functools

import jax
import jax.numpy as jnp
from jax import lax
from jax.experimental import pallas as pl
from jax.experimental.pallas import tpu as pltpu
from jax.experimental.pallas import tpu_sc as plsc

USER_NUM = 60000
ITEM_NUM = 40000
N_NODES = USER_NUM + ITEM_NUM
N_EDGES = 1600000
D = 32

NC = 2
NS = 16
NW = NC * NS
L = 16

HALF = N_NODES // NC
TRASH = HALF
ACC_ROWS = 50048

OCH = 64
NZCH = ACC_ROWS // OCH
N_FULL_CHUNKS = HALF // OCH
REM_ROWS = HALF - N_FULL_CHUNKS * OCH

CHUNK = 128
GK = 16
GE = GK * CHUNK
CAPC = 12800
EPC = CAPC * CHUNK

SCN = N_EDGES // NW
SGRP = 2000
NSGRP = SCN // SGRP
FIFO = 4096


def _zero_vec():
  return jnp.zeros((L,), jnp.float32)


def _count_body(rows_h, counts_h, rbuf, cbuf):
  c = lax.axis_index("c")
  s = lax.axis_index("s")
  wid = c * NS + s

  def group(g, cnt):
    pltpu.sync_copy(rows_h.at[pl.ds(wid * SCN + g * SGRP, SGRP)], rbuf)

    def step(i, cnt2):
      rr = rbuf[pl.ds(i * L, L)]
      return cnt2 + jnp.where(rr < HALF, 1, 0).astype(jnp.int32)
    return lax.fori_loop(0, SGRP // L, step, cnt)

  cnt = lax.fori_loop(0, NSGRP, group, jnp.zeros((L,), jnp.int32))
  total = cnt[0]
  for _i in range(1, L):
    total = total + cnt[_i]
  lanes = lax.iota(jnp.int32, L)
  cbuf[pl.ds(0, L)] = jnp.where(lanes == 0, total, 0)
  pltpu.sync_copy(cbuf, counts_h.at[wid])


def _route_body(rows_h, cols_h, vals_h, counts_h,
                cols2, rows2, vals2, summary,
                rbuf, cbuf, vbuf, cntb, tb,
                fc0, fr0, fv0, fc1, fr1, fv1):
  c = lax.axis_index("c")
  s = lax.axis_index("s")
  wid = c * NS + s
  lanes = lax.iota(jnp.int32, L)

  pltpu.sync_copy(counts_h, cntb)
  base0 = jnp.int32(0)
  base1 = jnp.int32(0)
  ncc0 = jnp.int32(0)
  ncc1 = jnp.int32(0)
  for t in range(NW):
    cv = cntb[t, pl.ds(0, L)]
    c0 = cv[0]
    p0 = (c0 + (CHUNK - 1)) // CHUNK
    p1 = ((SCN - c0) + (CHUNK - 1)) // CHUNK
    is_before = jnp.int32(t) < wid
    base0 = base0 + jnp.where(is_before, p0, 0)
    base1 = base1 + jnp.where(is_before, p1, 0)
    ncc0 = ncc0 + p0
    ncc1 = ncc1 + p1

  fifos = ((fc0, fr0, fv0), (fc1, fr1, fv1))

  def flush(core, fo, dst_edge_off, n):
    fc, fr, fv = fifos[core]
    fo = pl.multiple_of(jnp.int32(fo), CHUNK)
    base = pl.multiple_of(core * EPC + dst_edge_off, CHUNK)
    pltpu.sync_copy(fc.at[pl.ds(fo, n)], cols2.at[pl.ds(base, n)])
    pltpu.sync_copy(fr.at[pl.ds(fo, n)], rows2.at[pl.ds(base, n)])
    pltpu.sync_copy(fv.at[pl.ds(fo, n)], vals2.at[pl.ds(base, n)])

  def scan_group(g, carry):
    off0, off1, fl0, fl1 = carry
    goff = wid * SCN + g * SGRP
    pltpu.sync_copy(rows_h.at[pl.ds(goff, SGRP)], rbuf)
    pltpu.sync_copy(cols_h.at[pl.ds(goff, SGRP)], cbuf)
    pltpu.sync_copy(vals_h.at[pl.ds(goff, SGRP)], vbuf)

    def step(i, carry2):
      o0, o1 = carry2
      rr = rbuf[pl.ds(i * L, L)]
      cc = cbuf[pl.ds(i * L, L)]
      vv = vbuf[pl.ds(i * L, L)]
      m0 = rr < HALF
      x = jnp.where(m0, 1, 0).astype(jnp.int32)
      for sh in (1, 2, 4, 8):
        shifted = x[jnp.maximum(lanes - sh, 0)]
        x = x + jnp.where(lanes >= sh, shifted, 0)
      n0 = x[L - 1]
      dv = lanes + 1
      lo = jnp.zeros((L,), jnp.int32)
      for stp in (8, 4, 2, 1):
        cand = lo + stp
        pc = x[cand - 1]
        lo = jnp.where(pc < dv, cand, lo)
      fc0[pl.ds(o0, L)] = cc[lo]
      fr0[pl.ds(o0, L)] = rr[lo]
      fv0[pl.ds(o0, L)] = vv[lo]
      q = dv - x
      lo1 = jnp.zeros((L,), jnp.int32)
      for stp in (8, 4, 2, 1):
        cand = lo1 + stp
        pc = q[cand - 1]
        lo1 = jnp.where(pc < dv, cand, lo1)
      fc1[pl.ds(o1, L)] = cc[lo1]
      fr1[pl.ds(o1, L)] = rr[lo1] - HALF
      fv1[pl.ds(o1, L)] = vv[lo1]
      return (o0 + n0, o1 + (L - n0))

    off0, off1 = lax.fori_loop(0, SGRP // L, step, (off0, off1))

    def do_flush(core, off, fl, base):
      full = off >= GE

      @pl.when(full)
      def _():
        flush(core, 0, base * CHUNK + fl, GE)
        fc, fr, fv = fifos[core]

        def mv(i, carry3):
          fc[pl.ds(i * L, L)] = fc[pl.ds(GE + i * L, L)]
          fr[pl.ds(i * L, L)] = fr[pl.ds(GE + i * L, L)]
          fv[pl.ds(i * L, L)] = fv[pl.ds(GE + i * L, L)]
          return carry3
        lax.fori_loop(0, GE // L, mv, 0)

      off = jnp.where(full, off - GE, off)
      fl = jnp.where(full, fl + GE, fl)
      return off, fl

    off0, fl0 = do_flush(0, off0, fl0, base0)
    off1, fl1 = do_flush(1, off1, fl1, base1)
    return (off0, off1, fl0, fl1)

  off0, off1, fl0, fl1 = lax.fori_loop(
      0, NSGRP, scan_group,
      (jnp.int32(0), jnp.int32(0), jnp.int32(0), jnp.int32(0)))

  def tail(core, off, fl, base):
    fc, fr, fv = fifos[core]
    pstart = off // L

    def padv(i, carry2):
      b = (pstart + i) * L
      idxv = lanes + b
      m = idxv >= off
      cvv = fc[pl.ds(b, L)]
      rvv = fr[pl.ds(b, L)]
      vvv = fv[pl.ds(b, L)]
      fc[pl.ds(b, L)] = jnp.where(m, 0, cvv)
      fr[pl.ds(b, L)] = jnp.where(m, TRASH, rvv)
      fv[pl.ds(b, L)] = jnp.where(m, 0.0, vvv)
      return carry2
    lax.fori_loop(0, (CHUNK // L) + 1, padv, 0)

    rem_ch = (off + (CHUNK - 1)) // CHUNK
    fo = jnp.int32(0)
    for nch in (16, 8, 4, 2, 1):
      cond = (rem_ch & nch) != 0
      n = nch * CHUNK
      fo_now = fo

      @pl.when(cond)
      def _(core=core, fo_now=fo_now, n=n):
        flush(core, fo_now, base * CHUNK + fl + fo_now, n)
      fo = fo + jnp.where(cond, n, 0)

  tail(0, off0, fl0, base0)
  tail(1, off1, fl1, base1)

  def fill_const(ref, val):
    def f(i, carry2):
      ref[pl.ds(i * L, L)] = jnp.full((L,), val, ref.dtype)
      return carry2
    lax.fori_loop(0, CHUNK // L, f, 0)
  fill_const(fc0, 0)
  fill_const(fr0, TRASH)
  fill_const(fv0, 0.0)

  def pad_region(core, ncc):
    pad_end = ((ncc + 255) // 256) * 256

    def padc(j, carry2):
      ch = ncc + wid + j * NW

      @pl.when(ch < pad_end)
      def _():
        base = pl.multiple_of(core * EPC + ch * CHUNK, CHUNK)
        pltpu.sync_copy(fc0.at[pl.ds(0, CHUNK)], cols2.at[pl.ds(base, CHUNK)])
        pltpu.sync_copy(fr0.at[pl.ds(0, CHUNK)], rows2.at[pl.ds(base, CHUNK)])
        pltpu.sync_copy(fv0.at[pl.ds(0, CHUNK)], vals2.at[pl.ds(base, CHUNK)])
      return carry2
    lax.fori_loop(0, 8, padc, 0)

  pad_region(0, ncc0)
  pad_region(1, ncc1)

  @pl.when(wid == 0)
  def _():
    tb[pl.ds(0, L)] = jnp.where(lanes == 0, ncc0,
                                jnp.where(lanes == 1, ncc1, 0))
    pltpu.sync_copy(tb, summary.at[pl.ds(0, L)])


def _layer_body(finalize, ego_h, cols2, rows2, vals2, summary_h, e1_h, out_h,
                acc, obuf, b1, b2, sumv, ecol, erowl, evalv, radj,
                gath0, gath1, gath2, sem0, sem1, sem2, ssem0, ssem1, ssem2):
  c = lax.axis_index("c")
  s = lax.axis_index("s")
  base_row = c * HALF

  pltpu.sync_copy(summary_h, sumv)
  sv = sumv[pl.ds(0, L)]
  ncc = jnp.where(c == 0, sv[0], sv[1])
  ngrp = (ncc + 255) // 256

  def zfill(i, carry):
    obuf[i, pl.ds(0, L)] = _zero_vec()
    obuf[i, pl.ds(L, L)] = _zero_vec()
    return carry
  lax.fori_loop(0, OCH, zfill, 0)

  def zcopy(j, carry):
    cid = s + j * NS
    @pl.when(cid < NZCH)
    def _():
      pltpu.sync_copy(obuf, acc.at[pl.ds(cid * OCH, OCH)])
    return carry
  lax.fori_loop(0, (NZCH + NS - 1) // NS, zcopy, 0)
  plsc.subcore_barrier()

  def group_step(g, carry):
    goff = c * EPC + (s * ngrp + g) * GE
    pltpu.sync_copy(cols2.at[pl.ds(goff, GE)], ecol)
    desc = pltpu.async_copy(ego_h.at[ecol.at[pl.ds(0, CHUNK)]], gath0, sem0)
    pltpu.sync_copy(rows2.at[pl.ds(goff, GE)], erowl)
    pltpu.sync_copy(vals2.at[pl.ds(goff, GE)], evalv)

    def rcopy_k(k, carry2):
      def rcopy_i(i, carry3):
        radj[k, pl.ds(i * L, L)] = erowl[pl.ds(k * CHUNK + i * L, L)]
        return carry3
      lax.fori_loop(0, CHUNK // L, rcopy_i, 0)
      return carry2
    lax.fori_loop(0, GK, rcopy_k, 0)

    bufs = (gath0, gath1, gath2)
    gsems = (sem0, sem1, sem2)
    ssems = (ssem0, ssem1, ssem2)
    sdescs = [None] * GK
    for k in range(GK):
      gbuf = bufs[k % 3]
      desc.wait()
      if k + 1 < GK:
        if k >= 2:
          sdescs[k - 2].wait()
        desc = pltpu.async_copy(
            ego_h.at[ecol.at[pl.ds((k + 1) * CHUNK, CHUNK)]],
            bufs[(k + 1) % 3], gsems[(k + 1) % 3])

      def scale_g(i, carry2):
        vg = evalv[pl.ds(k * CHUNK + i * L, L)]
        for lane in range(L):
          e = i * L + lane
          b = jnp.full((L,), vg[lane], jnp.float32)
          gbuf[e, pl.ds(0, L)] = gbuf[e, pl.ds(0, L)] * b
          gbuf[e, pl.ds(L, L)] = gbuf[e, pl.ds(L, L)] * b
        return carry2
      lax.fori_loop(0, CHUNK // L, scale_g, 0, unroll=2)

      sdescs[k] = pltpu.async_copy(
          gbuf, acc.at[radj.at[k]], ssems[k % 3], add=True)
    for k in range(GK - 3, GK):
      sdescs[k].wait()
    return carry
  lax.fori_loop(0, ngrp, group_step, 0)
  plsc.subcore_barrier()

  def emit_chunk(r0, n):
    pltpu.sync_copy(acc.at[pl.ds(r0, n)], obuf.at[pl.ds(0, n)])
    if finalize:
      pltpu.sync_copy(e1_h.at[pl.ds(base_row + r0, n)], b1.at[pl.ds(0, n)])
      pltpu.sync_copy(ego_h.at[pl.ds(base_row + r0, n)], b2.at[pl.ds(0, n)])

      def mean_row(i, carry2):
        third = jnp.full((L,), 1.0 / 3.0, jnp.float32)
        lo = (obuf[i, pl.ds(0, L)] + b1[i, pl.ds(0, L)] + b2[i, pl.ds(0, L)])
        hi = (obuf[i, pl.ds(L, L)] + b1[i, pl.ds(L, L)] + b2[i, pl.ds(L, L)])
        obuf[i, pl.ds(0, L)] = lo * third
        obuf[i, pl.ds(L, L)] = hi * third
        return carry2
      lax.fori_loop(0, n, mean_row, 0)
    pltpu.sync_copy(obuf.at[pl.ds(0, n)], out_h.at[pl.ds(base_row + r0, n)])

  def cout(j, carry):
    cid = s + j * NS
    @pl.when(cid < N_FULL_CHUNKS)
    def _():
      emit_chunk(cid * OCH, OCH)
    return carry
  lax.fori_loop(0, (N_FULL_CHUNKS + NS - 1) // NS, cout, 0)

  @pl.when(s == NS - 1)
  def _():
    emit_chunk(N_FULL_CHUNKS * OCH, REM_ROWS)


def _sc_mesh():
  return plsc.VectorSubcoreMesh(core_axis_name="c", subcore_axis_name="s")


def _make_count():
  return pl.kernel(
      _count_body,
      out_type=jax.ShapeDtypeStruct((NW, L), jnp.int32),
      mesh=_sc_mesh(),
      scratch_types=[
          pltpu.VMEM((SGRP,), jnp.int32),
          pltpu.VMEM((L,), jnp.int32),
      ],
      compiler_params=pltpu.CompilerParams(use_tc_tiling_on_sc=False),
      name="lightgcn_count",
  )


def _make_route():
  return pl.kernel(
      _route_body,
      out_type=(
          jax.ShapeDtypeStruct((NC * EPC,), jnp.int32),
          jax.ShapeDtypeStruct((NC * EPC,), jnp.int32),
          jax.ShapeDtypeStruct((NC * EPC,), jnp.float32),
          jax.ShapeDtypeStruct((L,), jnp.int32),
      ),
      mesh=_sc_mesh(),
      scratch_types=[
          pltpu.VMEM((SGRP,), jnp.int32),
          pltpu.VMEM((SGRP,), jnp.int32),
          pltpu.VMEM((SGRP,), jnp.float32),
          pltpu.VMEM((NW, L), jnp.int32),
          pltpu.VMEM((L,), jnp.int32),
          pltpu.VMEM((FIFO,), jnp.int32),
          pltpu.VMEM((FIFO,), jnp.int32),
          pltpu.VMEM((FIFO,), jnp.float32),
          pltpu.VMEM((FIFO,), jnp.int32),
          pltpu.VMEM((FIFO,), jnp.int32),
          pltpu.VMEM((FIFO,), jnp.float32),
      ],
      compiler_params=pltpu.CompilerParams(use_tc_tiling_on_sc=False),
      name="lightgcn_route",
  )


def _make_layer(finalize):
  return pl.kernel(
      functools.partial(_layer_body, finalize),
      out_type=jax.ShapeDtypeStruct((N_NODES, D), jnp.float32),
      mesh=_sc_mesh(),
      scratch_types=[
          pltpu.VMEM_SHARED((ACC_ROWS, D), jnp.float32),
          pltpu.VMEM((OCH, D), jnp.float32),
          pltpu.VMEM((OCH, D), jnp.float32),
          pltpu.VMEM((OCH, D), jnp.float32),
          pltpu.VMEM((L,), jnp.int32),
          pltpu.VMEM((GE,), jnp.int32),
          pltpu.VMEM((GE,), jnp.int32),
          pltpu.VMEM((GE,), jnp.float32),
          pltpu.VMEM((GK, CHUNK), jnp.int32),
          pltpu.VMEM((CHUNK, D), jnp.float32),
          pltpu.VMEM((CHUNK, D), jnp.float32),
          pltpu.VMEM((CHUNK, D), jnp.float32),
          pltpu.SemaphoreType.DMA,
          pltpu.SemaphoreType.DMA,
          pltpu.SemaphoreType.DMA,
          pltpu.SemaphoreType.DMA,
          pltpu.SemaphoreType.DMA,
          pltpu.SemaphoreType.DMA,
      ],
      compiler_params=pltpu.CompilerParams(use_tc_tiling_on_sc=False),
      name="lightgcn_layer_final" if finalize else "lightgcn_layer",
  )


def kernel(user_emb, item_emb, adj_indices, adj_values):
  ego0 = jnp.concatenate([user_emb, item_emb], axis=0)
  rows = adj_indices[0].astype(jnp.int32)
  cols = adj_indices[1].astype(jnp.int32)
  vals = adj_values.astype(jnp.float32)

  counts = _make_count()(rows)
  cols2, rows2, vals2, summary = _make_route()(rows, cols, vals, counts)

  layer = _make_layer(False)
  layer_final = _make_layer(True)

  dummy = jnp.zeros((8, D), jnp.float32)
  e1 = layer(ego0, cols2, rows2, vals2, summary, dummy)
  e2 = layer(e1, cols2, rows2, vals2, summary, dummy)
  out = layer_final(e2, cols2, rows2, vals2, summary, e1)
  return (out[:USER_NUM], out[USER_NUM:])


# speedup vs baseline: 1.0156x; 1.0156x over previous
import functools

import jax
import jax.numpy as jnp
from jax import lax
from jax.experimental import pallas as pl
from jax.experimental.pallas import tpu as pltpu
from jax.experimental.pallas import tpu_sc as plsc

USER_NUM = 60000
ITEM_NUM = 40000
N_NODES = USER_NUM + ITEM_NUM
N_EDGES = 1600000
D = 32

NC = 2
NS = 16
NW = NC * NS
L = 16

HALF = N_NODES // NC
TRASH = HALF
ACC_ROWS = 50048

OCH = 128
NZCH = ACC_ROWS // OCH
N_FULL_CHUNKS = HALF // OCH
REM_ROWS = HALF - N_FULL_CHUNKS * OCH

CHUNK = 128
GK = 16
GE = GK * CHUNK
CAPC = 12800
EPC = CAPC * CHUNK

SCN = N_EDGES // NW
SGRP = 2000
NSGRP = SCN // SGRP
FIFO = 4096


def _zero_vec():
  return jnp.zeros((L,), jnp.float32)


def _count_body(rows_h, counts_h, rbuf, cbuf):
  c = lax.axis_index("c")
  s = lax.axis_index("s")
  wid = c * NS + s

  def group(g, cnt):
    pltpu.sync_copy(rows_h.at[pl.ds(wid * SCN + g * SGRP, SGRP)], rbuf)

    def step(i, cnt2):
      rr = rbuf[pl.ds(i * L, L)]
      return cnt2 + jnp.where(rr < HALF, 1, 0).astype(jnp.int32)
    return lax.fori_loop(0, SGRP // L, step, cnt)

  cnt = lax.fori_loop(0, NSGRP, group, jnp.zeros((L,), jnp.int32))
  total = cnt[0]
  for _i in range(1, L):
    total = total + cnt[_i]
  lanes = lax.iota(jnp.int32, L)
  cbuf[pl.ds(0, L)] = jnp.where(lanes == 0, total, 0)
  pltpu.sync_copy(cbuf, counts_h.at[wid])


def _route_body(rows_h, cols_h, vals_h, counts_h,
                cols2, rows2, vals2, summary,
                rbuf, cbuf, vbuf, cntb, tb,
                fc0, fr0, fv0, fc1, fr1, fv1):
  c = lax.axis_index("c")
  s = lax.axis_index("s")
  wid = c * NS + s
  lanes = lax.iota(jnp.int32, L)

  pltpu.sync_copy(counts_h, cntb)
  base0 = jnp.int32(0)
  base1 = jnp.int32(0)
  ncc0 = jnp.int32(0)
  ncc1 = jnp.int32(0)
  for t in range(NW):
    cv = cntb[t, pl.ds(0, L)]
    c0 = cv[0]
    p0 = (c0 + (CHUNK - 1)) // CHUNK
    p1 = ((SCN - c0) + (CHUNK - 1)) // CHUNK
    is_before = jnp.int32(t) < wid
    base0 = base0 + jnp.where(is_before, p0, 0)
    base1 = base1 + jnp.where(is_before, p1, 0)
    ncc0 = ncc0 + p0
    ncc1 = ncc1 + p1

  fifos = ((fc0, fr0, fv0), (fc1, fr1, fv1))

  def flush(core, fo, dst_edge_off, n):
    fc, fr, fv = fifos[core]
    fo = pl.multiple_of(jnp.int32(fo), CHUNK)
    base = pl.multiple_of(core * EPC + dst_edge_off, CHUNK)
    pltpu.sync_copy(fc.at[pl.ds(fo, n)], cols2.at[pl.ds(base, n)])
    pltpu.sync_copy(fr.at[pl.ds(fo, n)], rows2.at[pl.ds(base, n)])
    pltpu.sync_copy(fv.at[pl.ds(fo, n)], vals2.at[pl.ds(base, n)])

  def scan_group(g, carry):
    off0, off1, fl0, fl1 = carry
    goff = wid * SCN + g * SGRP
    pltpu.sync_copy(rows_h.at[pl.ds(goff, SGRP)], rbuf)
    pltpu.sync_copy(cols_h.at[pl.ds(goff, SGRP)], cbuf)
    pltpu.sync_copy(vals_h.at[pl.ds(goff, SGRP)], vbuf)

    def step(i, carry2):
      o0, o1 = carry2
      rr = rbuf[pl.ds(i * L, L)]
      cc = cbuf[pl.ds(i * L, L)]
      vv = vbuf[pl.ds(i * L, L)]
      m0 = rr < HALF
      x = jnp.where(m0, 1, 0).astype(jnp.int32)
      for sh in (1, 2, 4, 8):
        shifted = x[jnp.maximum(lanes - sh, 0)]
        x = x + jnp.where(lanes >= sh, shifted, 0)
      n0 = x[L - 1]
      dv = lanes + 1
      lo = jnp.zeros((L,), jnp.int32)
      for stp in (8, 4, 2, 1):
        cand = lo + stp
        pc = x[cand - 1]
        lo = jnp.where(pc < dv, cand, lo)
      fc0[pl.ds(o0, L)] = cc[lo]
      fr0[pl.ds(o0, L)] = rr[lo]
      fv0[pl.ds(o0, L)] = vv[lo]
      q = dv - x
      lo1 = jnp.zeros((L,), jnp.int32)
      for stp in (8, 4, 2, 1):
        cand = lo1 + stp
        pc = q[cand - 1]
        lo1 = jnp.where(pc < dv, cand, lo1)
      fc1[pl.ds(o1, L)] = cc[lo1]
      fr1[pl.ds(o1, L)] = rr[lo1] - HALF
      fv1[pl.ds(o1, L)] = vv[lo1]
      return (o0 + n0, o1 + (L - n0))

    off0, off1 = lax.fori_loop(0, SGRP // L, step, (off0, off1))

    def do_flush(core, off, fl, base):
      full = off >= GE

      @pl.when(full)
      def _():
        flush(core, 0, base * CHUNK + fl, GE)
        fc, fr, fv = fifos[core]

        def mv(i, carry3):
          fc[pl.ds(i * L, L)] = fc[pl.ds(GE + i * L, L)]
          fr[pl.ds(i * L, L)] = fr[pl.ds(GE + i * L, L)]
          fv[pl.ds(i * L, L)] = fv[pl.ds(GE + i * L, L)]
          return carry3
        lax.fori_loop(0, GE // L, mv, 0)

      off = jnp.where(full, off - GE, off)
      fl = jnp.where(full, fl + GE, fl)
      return off, fl

    off0, fl0 = do_flush(0, off0, fl0, base0)
    off1, fl1 = do_flush(1, off1, fl1, base1)
    return (off0, off1, fl0, fl1)

  off0, off1, fl0, fl1 = lax.fori_loop(
      0, NSGRP, scan_group,
      (jnp.int32(0), jnp.int32(0), jnp.int32(0), jnp.int32(0)))

  def tail(core, off, fl, base):
    fc, fr, fv = fifos[core]
    pstart = off // L

    def padv(i, carry2):
      b = (pstart + i) * L
      idxv = lanes + b
      m = idxv >= off
      cvv = fc[pl.ds(b, L)]
      rvv = fr[pl.ds(b, L)]
      vvv = fv[pl.ds(b, L)]
      fc[pl.ds(b, L)] = jnp.where(m, 0, cvv)
      fr[pl.ds(b, L)] = jnp.where(m, TRASH, rvv)
      fv[pl.ds(b, L)] = jnp.where(m, 0.0, vvv)
      return carry2
    lax.fori_loop(0, (CHUNK // L) + 1, padv, 0)

    rem_ch = (off + (CHUNK - 1)) // CHUNK
    fo = jnp.int32(0)
    for nch in (16, 8, 4, 2, 1):
      cond = (rem_ch & nch) != 0
      n = nch * CHUNK
      fo_now = fo

      @pl.when(cond)
      def _(core=core, fo_now=fo_now, n=n):
        flush(core, fo_now, base * CHUNK + fl + fo_now, n)
      fo = fo + jnp.where(cond, n, 0)

  tail(0, off0, fl0, base0)
  tail(1, off1, fl1, base1)

  def fill_const(ref, val):
    def f(i, carry2):
      ref[pl.ds(i * L, L)] = jnp.full((L,), val, ref.dtype)
      return carry2
    lax.fori_loop(0, CHUNK // L, f, 0)
  fill_const(fc0, 0)
  fill_const(fr0, TRASH)
  fill_const(fv0, 0.0)

  def pad_region(core, ncc):
    pad_end = ((ncc + 255) // 256) * 256

    def padc(j, carry2):
      ch = ncc + wid + j * NW

      @pl.when(ch < pad_end)
      def _():
        base = pl.multiple_of(core * EPC + ch * CHUNK, CHUNK)
        pltpu.sync_copy(fc0.at[pl.ds(0, CHUNK)], cols2.at[pl.ds(base, CHUNK)])
        pltpu.sync_copy(fr0.at[pl.ds(0, CHUNK)], rows2.at[pl.ds(base, CHUNK)])
        pltpu.sync_copy(fv0.at[pl.ds(0, CHUNK)], vals2.at[pl.ds(base, CHUNK)])
      return carry2
    lax.fori_loop(0, 8, padc, 0)

  pad_region(0, ncc0)
  pad_region(1, ncc1)

  @pl.when(wid == 0)
  def _():
    tb[pl.ds(0, L)] = jnp.where(lanes == 0, ncc0,
                                jnp.where(lanes == 1, ncc1, 0))
    pltpu.sync_copy(tb, summary.at[pl.ds(0, L)])


def _layer_body(finalize, ego_h, cols2, rows2, vals2, summary_h, e1_h, out_h,
                acc, sumv, ecol, erowl, evalv, radj,
                gath0, gath1, gath2, sem0, sem1, sem2, ssem0, ssem1, ssem2):
  obuf, b1, b2 = gath0, gath1, gath2
  c = lax.axis_index("c")
  s = lax.axis_index("s")
  base_row = c * HALF

  pltpu.sync_copy(summary_h, sumv)
  sv = sumv[pl.ds(0, L)]
  ncc = jnp.where(c == 0, sv[0], sv[1])
  ngrp = (ncc + 255) // 256

  def zfill(i, carry):
    obuf[i, pl.ds(0, L)] = _zero_vec()
    obuf[i, pl.ds(L, L)] = _zero_vec()
    return carry
  lax.fori_loop(0, OCH, zfill, 0)

  def zcopy(j, carry):
    cid = s + j * NS
    @pl.when(cid < NZCH)
    def _():
      pltpu.sync_copy(obuf, acc.at[pl.ds(cid * OCH, OCH)])
    return carry
  lax.fori_loop(0, (NZCH + NS - 1) // NS, zcopy, 0)
  plsc.subcore_barrier()

  def group_step(g, carry):
    goff = c * EPC + (s * ngrp + g) * GE
    pltpu.sync_copy(cols2.at[pl.ds(goff, GE)], ecol)
    desc = pltpu.async_copy(ego_h.at[ecol.at[pl.ds(0, CHUNK)]], gath0, sem0)
    pltpu.sync_copy(rows2.at[pl.ds(goff, GE)], erowl)
    pltpu.sync_copy(vals2.at[pl.ds(goff, GE)], evalv)

    def rcopy_k(k, carry2):
      def rcopy_i(i, carry3):
        radj[k, pl.ds(i * L, L)] = erowl[pl.ds(k * CHUNK + i * L, L)]
        return carry3
      lax.fori_loop(0, CHUNK // L, rcopy_i, 0)
      return carry2
    lax.fori_loop(0, GK, rcopy_k, 0)

    bufs = (gath0, gath1, gath2)
    gsems = (sem0, sem1, sem2)
    ssems = (ssem0, ssem1, ssem2)
    sdescs = [None] * GK
    for k in range(GK):
      gbuf = bufs[k % 3]
      desc.wait()
      if k + 1 < GK:
        if k >= 2:
          sdescs[k - 2].wait()
        desc = pltpu.async_copy(
            ego_h.at[ecol.at[pl.ds((k + 1) * CHUNK, CHUNK)]],
            bufs[(k + 1) % 3], gsems[(k + 1) % 3])

      def scale_g(i, carry2):
        vg = evalv[pl.ds(k * CHUNK + i * L, L)]
        for lane in range(L):
          e = i * L + lane
          b = jnp.full((L,), vg[lane], jnp.float32)
          gbuf[e, pl.ds(0, L)] = gbuf[e, pl.ds(0, L)] * b
          gbuf[e, pl.ds(L, L)] = gbuf[e, pl.ds(L, L)] * b
        return carry2
      lax.fori_loop(0, CHUNK // L, scale_g, 0, unroll=2)

      sdescs[k] = pltpu.async_copy(
          gbuf, acc.at[radj.at[k]], ssems[k % 3], add=True)
    for k in range(GK - 3, GK):
      sdescs[k].wait()
    return carry
  lax.fori_loop(0, ngrp, group_step, 0)
  plsc.subcore_barrier()

  def emit_chunk(r0, n):
    pltpu.sync_copy(acc.at[pl.ds(r0, n)], obuf.at[pl.ds(0, n)])
    if finalize:
      pltpu.sync_copy(e1_h.at[pl.ds(base_row + r0, n)], b1.at[pl.ds(0, n)])
      pltpu.sync_copy(ego_h.at[pl.ds(base_row + r0, n)], b2.at[pl.ds(0, n)])

      def mean_row(i, carry2):
        third = jnp.full((L,), 1.0 / 3.0, jnp.float32)
        lo = (obuf[i, pl.ds(0, L)] + b1[i, pl.ds(0, L)] + b2[i, pl.ds(0, L)])
        hi = (obuf[i, pl.ds(L, L)] + b1[i, pl.ds(L, L)] + b2[i, pl.ds(L, L)])
        obuf[i, pl.ds(0, L)] = lo * third
        obuf[i, pl.ds(L, L)] = hi * third
        return carry2
      lax.fori_loop(0, n, mean_row, 0)
    pltpu.sync_copy(obuf.at[pl.ds(0, n)], out_h.at[pl.ds(base_row + r0, n)])

  def cout(j, carry):
    cid = s + j * NS
    @pl.when(cid < N_FULL_CHUNKS)
    def _():
      emit_chunk(cid * OCH, OCH)
    return carry
  lax.fori_loop(0, (N_FULL_CHUNKS + NS - 1) // NS, cout, 0)

  @pl.when(s == NS - 1)
  def _():
    emit_chunk(N_FULL_CHUNKS * OCH, REM_ROWS)


def _sc_mesh():
  return plsc.VectorSubcoreMesh(core_axis_name="c", subcore_axis_name="s")


def _make_count():
  return pl.kernel(
      _count_body,
      out_type=jax.ShapeDtypeStruct((NW, L), jnp.int32),
      mesh=_sc_mesh(),
      scratch_types=[
          pltpu.VMEM((SGRP,), jnp.int32),
          pltpu.VMEM((L,), jnp.int32),
      ],
      compiler_params=pltpu.CompilerParams(use_tc_tiling_on_sc=False),
      name="lightgcn_count",
  )


def _make_route():
  return pl.kernel(
      _route_body,
      out_type=(
          jax.ShapeDtypeStruct((NC * EPC,), jnp.int32),
          jax.ShapeDtypeStruct((NC * EPC,), jnp.int32),
          jax.ShapeDtypeStruct((NC * EPC,), jnp.float32),
          jax.ShapeDtypeStruct((L,), jnp.int32),
      ),
      mesh=_sc_mesh(),
      scratch_types=[
          pltpu.VMEM((SGRP,), jnp.int32),
          pltpu.VMEM((SGRP,), jnp.int32),
          pltpu.VMEM((SGRP,), jnp.float32),
          pltpu.VMEM((NW, L), jnp.int32),
          pltpu.VMEM((L,), jnp.int32),
          pltpu.VMEM((FIFO,), jnp.int32),
          pltpu.VMEM((FIFO,), jnp.int32),
          pltpu.VMEM((FIFO,), jnp.float32),
          pltpu.VMEM((FIFO,), jnp.int32),
          pltpu.VMEM((FIFO,), jnp.int32),
          pltpu.VMEM((FIFO,), jnp.float32),
      ],
      compiler_params=pltpu.CompilerParams(use_tc_tiling_on_sc=False),
      name="lightgcn_route",
  )


def _make_layer(finalize):
  return pl.kernel(
      functools.partial(_layer_body, finalize),
      out_type=jax.ShapeDtypeStruct((N_NODES, D), jnp.float32),
      mesh=_sc_mesh(),
      scratch_types=[
          pltpu.VMEM_SHARED((ACC_ROWS, D), jnp.float32),
          pltpu.VMEM((L,), jnp.int32),
          pltpu.VMEM((GE,), jnp.int32),
          pltpu.VMEM((GE,), jnp.int32),
          pltpu.VMEM((GE,), jnp.float32),
          pltpu.VMEM((GK, CHUNK), jnp.int32),
          pltpu.VMEM((CHUNK, D), jnp.float32),
          pltpu.VMEM((CHUNK, D), jnp.float32),
          pltpu.VMEM((CHUNK, D), jnp.float32),
          pltpu.SemaphoreType.DMA,
          pltpu.SemaphoreType.DMA,
          pltpu.SemaphoreType.DMA,
          pltpu.SemaphoreType.DMA,
          pltpu.SemaphoreType.DMA,
          pltpu.SemaphoreType.DMA,
      ],
      compiler_params=pltpu.CompilerParams(use_tc_tiling_on_sc=False),
      name="lightgcn_layer_final" if finalize else "lightgcn_layer",
  )


def kernel(user_emb, item_emb, adj_indices, adj_values):
  ego0 = jnp.concatenate([user_emb, item_emb], axis=0)
  rows = adj_indices[0].astype(jnp.int32)
  cols = adj_indices[1].astype(jnp.int32)
  vals = adj_values.astype(jnp.float32)

  counts = _make_count()(rows)
  cols2, rows2, vals2, summary = _make_route()(rows, cols, vals, counts)

  layer = _make_layer(False)
  layer_final = _make_layer(True)

  dummy = jnp.zeros((8, D), jnp.float32)
  e1 = layer(ego0, cols2, rows2, vals2, summary, dummy)
  e2 = layer(e1, cols2, rows2, vals2, summary, dummy)
  out = layer_final(e2, cols2, rows2, vals2, summary, e1)
  return (out[:USER_NUM], out[USER_NUM:])


# speedup vs baseline: 1.0230x; 1.0073x over previous
import functools

import jax
import jax.numpy as jnp
from jax import lax
from jax.experimental import pallas as pl
from jax.experimental.pallas import tpu as pltpu
from jax.experimental.pallas import tpu_sc as plsc

USER_NUM = 60000
ITEM_NUM = 40000
N_NODES = USER_NUM + ITEM_NUM
N_EDGES = 1600000
D = 32

NC = 2
NS = 16
NW = NC * NS
L = 16

HALF = N_NODES // NC
TRASH = HALF
ACC_ROWS = 50048

OCH = 128
NZCH = ACC_ROWS // OCH
N_FULL_CHUNKS = HALF // OCH
REM_ROWS = HALF - N_FULL_CHUNKS * OCH

CHUNK = 128
GK = 16
GE = GK * CHUNK
CAPC = 12800
EPC = CAPC * CHUNK

SCN = N_EDGES // NW
SGRP = 2000
NSGRP = SCN // SGRP
FIFO = 4096


def _zero_vec():
  return jnp.zeros((L,), jnp.float32)


def _count_body(rows_h, counts_h, rbuf, cbuf):
  c = lax.axis_index("c")
  s = lax.axis_index("s")
  wid = c * NS + s

  def group(g, cnt):
    pltpu.sync_copy(rows_h.at[pl.ds(wid * SCN + g * SGRP, SGRP)], rbuf)

    def step(i, cnt2):
      rr = rbuf[pl.ds(i * L, L)]
      return cnt2 + jnp.where(rr < HALF, 1, 0).astype(jnp.int32)
    return lax.fori_loop(0, SGRP // L, step, cnt)

  cnt = lax.fori_loop(0, NSGRP, group, jnp.zeros((L,), jnp.int32))
  total = cnt[0]
  for _i in range(1, L):
    total = total + cnt[_i]
  lanes = lax.iota(jnp.int32, L)
  cbuf[pl.ds(0, L)] = jnp.where(lanes == 0, total, 0)
  pltpu.sync_copy(cbuf, counts_h.at[wid])


def _route_body(rows_h, cols_h, vals_h, counts_h,
                cols2, rows2, vals2, summary,
                rbuf, cbuf, vbuf, cntb, tb,
                fc0, fr0, fv0, fc1, fr1, fv1):
  c = lax.axis_index("c")
  s = lax.axis_index("s")
  wid = c * NS + s
  lanes = lax.iota(jnp.int32, L)

  pltpu.sync_copy(counts_h, cntb)
  base0 = jnp.int32(0)
  base1 = jnp.int32(0)
  ncc0 = jnp.int32(0)
  ncc1 = jnp.int32(0)
  for t in range(NW):
    cv = cntb[t, pl.ds(0, L)]
    c0 = cv[0]
    p0 = (c0 + (CHUNK - 1)) // CHUNK
    p1 = ((SCN - c0) + (CHUNK - 1)) // CHUNK
    is_before = jnp.int32(t) < wid
    base0 = base0 + jnp.where(is_before, p0, 0)
    base1 = base1 + jnp.where(is_before, p1, 0)
    ncc0 = ncc0 + p0
    ncc1 = ncc1 + p1

  fifos = ((fc0, fr0, fv0), (fc1, fr1, fv1))

  def flush(core, fo, dst_edge_off, n):
    fc, fr, fv = fifos[core]
    fo = pl.multiple_of(jnp.int32(fo), CHUNK)
    base = pl.multiple_of(core * EPC + dst_edge_off, CHUNK)
    pltpu.sync_copy(fc.at[pl.ds(fo, n)], cols2.at[pl.ds(base, n)])
    pltpu.sync_copy(fr.at[pl.ds(fo, n)], rows2.at[pl.ds(base, n)])
    pltpu.sync_copy(fv.at[pl.ds(fo, n)], vals2.at[pl.ds(base, n)])

  def scan_group(g, carry):
    off0, off1, fl0, fl1 = carry
    goff = wid * SCN + g * SGRP
    pltpu.sync_copy(rows_h.at[pl.ds(goff, SGRP)], rbuf)
    pltpu.sync_copy(cols_h.at[pl.ds(goff, SGRP)], cbuf)
    pltpu.sync_copy(vals_h.at[pl.ds(goff, SGRP)], vbuf)

    def step(i, carry2):
      o0, o1 = carry2
      rr = rbuf[pl.ds(i * L, L)]
      cc = cbuf[pl.ds(i * L, L)]
      vv = vbuf[pl.ds(i * L, L)]
      m0 = rr < HALF
      x = jnp.where(m0, 1, 0).astype(jnp.int32)
      for sh in (1, 2, 4, 8):
        shifted = x[jnp.maximum(lanes - sh, 0)]
        x = x + jnp.where(lanes >= sh, shifted, 0)
      n0 = x[L - 1]
      dv = lanes + 1
      lo = jnp.zeros((L,), jnp.int32)
      for stp in (8, 4, 2, 1):
        cand = lo + stp
        pc = x[cand - 1]
        lo = jnp.where(pc < dv, cand, lo)
      fc0[pl.ds(o0, L)] = cc[lo]
      fr0[pl.ds(o0, L)] = rr[lo]
      fv0[pl.ds(o0, L)] = vv[lo]
      q = dv - x
      lo1 = jnp.zeros((L,), jnp.int32)
      for stp in (8, 4, 2, 1):
        cand = lo1 + stp
        pc = q[cand - 1]
        lo1 = jnp.where(pc < dv, cand, lo1)
      fc1[pl.ds(o1, L)] = cc[lo1]
      fr1[pl.ds(o1, L)] = rr[lo1] - HALF
      fv1[pl.ds(o1, L)] = vv[lo1]
      return (o0 + n0, o1 + (L - n0))

    off0, off1 = lax.fori_loop(0, SGRP // L, step, (off0, off1))

    def do_flush(core, off, fl, base):
      full = off >= GE

      @pl.when(full)
      def _():
        flush(core, 0, base * CHUNK + fl, GE)
        fc, fr, fv = fifos[core]

        def mv(i, carry3):
          fc[pl.ds(i * L, L)] = fc[pl.ds(GE + i * L, L)]
          fr[pl.ds(i * L, L)] = fr[pl.ds(GE + i * L, L)]
          fv[pl.ds(i * L, L)] = fv[pl.ds(GE + i * L, L)]
          return carry3
        lax.fori_loop(0, GE // L, mv, 0)

      off = jnp.where(full, off - GE, off)
      fl = jnp.where(full, fl + GE, fl)
      return off, fl

    off0, fl0 = do_flush(0, off0, fl0, base0)
    off1, fl1 = do_flush(1, off1, fl1, base1)
    return (off0, off1, fl0, fl1)

  off0, off1, fl0, fl1 = lax.fori_loop(
      0, NSGRP, scan_group,
      (jnp.int32(0), jnp.int32(0), jnp.int32(0), jnp.int32(0)))

  def tail(core, off, fl, base):
    fc, fr, fv = fifos[core]
    pstart = off // L

    def padv(i, carry2):
      b = (pstart + i) * L
      idxv = lanes + b
      m = idxv >= off
      cvv = fc[pl.ds(b, L)]
      rvv = fr[pl.ds(b, L)]
      vvv = fv[pl.ds(b, L)]
      fc[pl.ds(b, L)] = jnp.where(m, 0, cvv)
      fr[pl.ds(b, L)] = jnp.where(m, TRASH, rvv)
      fv[pl.ds(b, L)] = jnp.where(m, 0.0, vvv)
      return carry2
    lax.fori_loop(0, (CHUNK // L) + 1, padv, 0)

    rem_ch = (off + (CHUNK - 1)) // CHUNK
    fo = jnp.int32(0)
    for nch in (16, 8, 4, 2, 1):
      cond = (rem_ch & nch) != 0
      n = nch * CHUNK
      fo_now = fo

      @pl.when(cond)
      def _(core=core, fo_now=fo_now, n=n):
        flush(core, fo_now, base * CHUNK + fl + fo_now, n)
      fo = fo + jnp.where(cond, n, 0)

  tail(0, off0, fl0, base0)
  tail(1, off1, fl1, base1)

  def fill_const(ref, val):
    def f(i, carry2):
      ref[pl.ds(i * L, L)] = jnp.full((L,), val, ref.dtype)
      return carry2
    lax.fori_loop(0, CHUNK // L, f, 0)
  fill_const(fc0, 0)
  fill_const(fr0, TRASH)
  fill_const(fv0, 0.0)

  def pad_region(core, ncc):
    pad_end = ((ncc + 255) // 256) * 256

    def padc(j, carry2):
      ch = ncc + wid + j * NW

      @pl.when(ch < pad_end)
      def _():
        base = pl.multiple_of(core * EPC + ch * CHUNK, CHUNK)
        pltpu.sync_copy(fc0.at[pl.ds(0, CHUNK)], cols2.at[pl.ds(base, CHUNK)])
        pltpu.sync_copy(fr0.at[pl.ds(0, CHUNK)], rows2.at[pl.ds(base, CHUNK)])
        pltpu.sync_copy(fv0.at[pl.ds(0, CHUNK)], vals2.at[pl.ds(base, CHUNK)])
      return carry2
    lax.fori_loop(0, 8, padc, 0)

  pad_region(0, ncc0)
  pad_region(1, ncc1)

  @pl.when(wid == 0)
  def _():
    tb[pl.ds(0, L)] = jnp.where(lanes == 0, ncc0,
                                jnp.where(lanes == 1, ncc1, 0))
    pltpu.sync_copy(tb, summary.at[pl.ds(0, L)])


def _layer_body(finalize, ego_h, cols2, rows2, vals2, summary_h, e1_h, out_h,
                acc, sumv, ecol, erowl, evalv, radj,
                gath0, gath1, gath2, gath3,
                sem0, sem1, sem2, sem3, ssem0, ssem1, ssem2, ssem3):
  obuf, b1, b2 = gath0, gath1, gath2
  c = lax.axis_index("c")
  s = lax.axis_index("s")
  base_row = c * HALF

  pltpu.sync_copy(summary_h, sumv)
  sv = sumv[pl.ds(0, L)]
  ncc = jnp.where(c == 0, sv[0], sv[1])
  ngrp = (ncc + 255) // 256

  def zfill(i, carry):
    obuf[i, pl.ds(0, L)] = _zero_vec()
    obuf[i, pl.ds(L, L)] = _zero_vec()
    return carry
  lax.fori_loop(0, OCH, zfill, 0)

  def zcopy(j, carry):
    cid = s + j * NS
    @pl.when(cid < NZCH)
    def _():
      pltpu.sync_copy(obuf, acc.at[pl.ds(cid * OCH, OCH)])
    return carry
  lax.fori_loop(0, (NZCH + NS - 1) // NS, zcopy, 0)
  plsc.subcore_barrier()

  def group_step(g, carry):
    goff = c * EPC + (s * ngrp + g) * GE
    pltpu.sync_copy(cols2.at[pl.ds(goff, GE)], ecol)
    desc0 = pltpu.async_copy(ego_h.at[ecol.at[pl.ds(0, CHUNK)]], gath0, sem0)
    desc1 = pltpu.async_copy(ego_h.at[ecol.at[pl.ds(CHUNK, CHUNK)]], gath1,
                             sem1)
    gdescs = [desc0, desc1]
    pltpu.sync_copy(rows2.at[pl.ds(goff, GE)], erowl)
    pltpu.sync_copy(vals2.at[pl.ds(goff, GE)], evalv)

    def rcopy_k(k, carry2):
      def rcopy_i(i, carry3):
        radj[k, pl.ds(i * L, L)] = erowl[pl.ds(k * CHUNK + i * L, L)]
        return carry3
      lax.fori_loop(0, CHUNK // L, rcopy_i, 0)
      return carry2
    lax.fori_loop(0, GK, rcopy_k, 0)

    bufs = (gath0, gath1, gath2, gath3)
    gsems = (sem0, sem1, sem2, sem3)
    ssems = (ssem0, ssem1, ssem2, ssem3)
    sdescs = [None] * GK
    for k in range(GK):
      gbuf = bufs[k % 4]
      gdescs[k].wait()
      if k + 2 < GK:
        if k >= 2:
          sdescs[k - 2].wait()
        gdescs.append(pltpu.async_copy(
            ego_h.at[ecol.at[pl.ds((k + 2) * CHUNK, CHUNK)]],
            bufs[(k + 2) % 4], gsems[(k + 2) % 4]))

      def scale_g(i, carry2):
        vg = evalv[pl.ds(k * CHUNK + i * L, L)]
        for lane in range(L):
          e = i * L + lane
          b = jnp.full((L,), vg[lane], jnp.float32)
          gbuf[e, pl.ds(0, L)] = gbuf[e, pl.ds(0, L)] * b
          gbuf[e, pl.ds(L, L)] = gbuf[e, pl.ds(L, L)] * b
        return carry2
      lax.fori_loop(0, CHUNK // L, scale_g, 0, unroll=2)

      sdescs[k] = pltpu.async_copy(
          gbuf, acc.at[radj.at[k]], ssems[k % 4], add=True)
    for k in range(GK - 4, GK):
      sdescs[k].wait()
    return carry
  lax.fori_loop(0, ngrp, group_step, 0)
  plsc.subcore_barrier()

  def emit_chunk(r0, n):
    pltpu.sync_copy(acc.at[pl.ds(r0, n)], obuf.at[pl.ds(0, n)])
    if finalize:
      pltpu.sync_copy(e1_h.at[pl.ds(base_row + r0, n)], b1.at[pl.ds(0, n)])
      pltpu.sync_copy(ego_h.at[pl.ds(base_row + r0, n)], b2.at[pl.ds(0, n)])

      def mean_row(i, carry2):
        third = jnp.full((L,), 1.0 / 3.0, jnp.float32)
        lo = (obuf[i, pl.ds(0, L)] + b1[i, pl.ds(0, L)] + b2[i, pl.ds(0, L)])
        hi = (obuf[i, pl.ds(L, L)] + b1[i, pl.ds(L, L)] + b2[i, pl.ds(L, L)])
        obuf[i, pl.ds(0, L)] = lo * third
        obuf[i, pl.ds(L, L)] = hi * third
        return carry2
      lax.fori_loop(0, n, mean_row, 0)
    pltpu.sync_copy(obuf.at[pl.ds(0, n)], out_h.at[pl.ds(base_row + r0, n)])

  def cout(j, carry):
    cid = s + j * NS
    @pl.when(cid < N_FULL_CHUNKS)
    def _():
      emit_chunk(cid * OCH, OCH)
    return carry
  lax.fori_loop(0, (N_FULL_CHUNKS + NS - 1) // NS, cout, 0)

  @pl.when(s == NS - 1)
  def _():
    emit_chunk(N_FULL_CHUNKS * OCH, REM_ROWS)


def _sc_mesh():
  return plsc.VectorSubcoreMesh(core_axis_name="c", subcore_axis_name="s")


def _make_count():
  return pl.kernel(
      _count_body,
      out_type=jax.ShapeDtypeStruct((NW, L), jnp.int32),
      mesh=_sc_mesh(),
      scratch_types=[
          pltpu.VMEM((SGRP,), jnp.int32),
          pltpu.VMEM((L,), jnp.int32),
      ],
      compiler_params=pltpu.CompilerParams(use_tc_tiling_on_sc=False),
      name="lightgcn_count",
  )


def _make_route():
  return pl.kernel(
      _route_body,
      out_type=(
          jax.ShapeDtypeStruct((NC * EPC,), jnp.int32),
          jax.ShapeDtypeStruct((NC * EPC,), jnp.int32),
          jax.ShapeDtypeStruct((NC * EPC,), jnp.float32),
          jax.ShapeDtypeStruct((L,), jnp.int32),
      ),
      mesh=_sc_mesh(),
      scratch_types=[
          pltpu.VMEM((SGRP,), jnp.int32),
          pltpu.VMEM((SGRP,), jnp.int32),
          pltpu.VMEM((SGRP,), jnp.float32),
          pltpu.VMEM((NW, L), jnp.int32),
          pltpu.VMEM((L,), jnp.int32),
          pltpu.VMEM((FIFO,), jnp.int32),
          pltpu.VMEM((FIFO,), jnp.int32),
          pltpu.VMEM((FIFO,), jnp.float32),
          pltpu.VMEM((FIFO,), jnp.int32),
          pltpu.VMEM((FIFO,), jnp.int32),
          pltpu.VMEM((FIFO,), jnp.float32),
      ],
      compiler_params=pltpu.CompilerParams(use_tc_tiling_on_sc=False),
      name="lightgcn_route",
  )


def _make_layer(finalize):
  return pl.kernel(
      functools.partial(_layer_body, finalize),
      out_type=jax.ShapeDtypeStruct((N_NODES, D), jnp.float32),
      mesh=_sc_mesh(),
      scratch_types=[
          pltpu.VMEM_SHARED((ACC_ROWS, D), jnp.float32),
          pltpu.VMEM((L,), jnp.int32),
          pltpu.VMEM((GE,), jnp.int32),
          pltpu.VMEM((GE,), jnp.int32),
          pltpu.VMEM((GE,), jnp.float32),
          pltpu.VMEM((GK, CHUNK), jnp.int32),
          pltpu.VMEM((CHUNK, D), jnp.float32),
          pltpu.VMEM((CHUNK, D), jnp.float32),
          pltpu.VMEM((CHUNK, D), jnp.float32),
          pltpu.VMEM((CHUNK, D), jnp.float32),
          pltpu.SemaphoreType.DMA,
          pltpu.SemaphoreType.DMA,
          pltpu.SemaphoreType.DMA,
          pltpu.SemaphoreType.DMA,
          pltpu.SemaphoreType.DMA,
          pltpu.SemaphoreType.DMA,
          pltpu.SemaphoreType.DMA,
          pltpu.SemaphoreType.DMA,
      ],
      compiler_params=pltpu.CompilerParams(use_tc_tiling_on_sc=False),
      name="lightgcn_layer_final" if finalize else "lightgcn_layer",
  )


def kernel(user_emb, item_emb, adj_indices, adj_values):
  ego0 = jnp.concatenate([user_emb, item_emb], axis=0)
  rows = adj_indices[0].astype(jnp.int32)
  cols = adj_indices[1].astype(jnp.int32)
  vals = adj_values.astype(jnp.float32)

  counts = _make_count()(rows)
  cols2, rows2, vals2, summary = _make_route()(rows, cols, vals, counts)

  layer = _make_layer(False)
  layer_final = _make_layer(True)

  dummy = jnp.zeros((8, D), jnp.float32)
  e1 = layer(ego0, cols2, rows2, vals2, summary, dummy)
  e2 = layer(e1, cols2, rows2, vals2, summary, dummy)
  out = layer_final(e2, cols2, rows2, vals2, summary, e1)
  return (out[:USER_NUM], out[USER_NUM:])


# speedup vs baseline: 1.2814x; 1.2526x over previous
import functools

import jax
import jax.numpy as jnp
from jax import lax
from jax.experimental import pallas as pl
from jax.experimental.pallas import tpu as pltpu
from jax.experimental.pallas import tpu_sc as plsc

USER_NUM = 60000
ITEM_NUM = 40000
N_NODES = USER_NUM + ITEM_NUM
N_EDGES = 1600000
D = 32

NC = 2
NS = 16
NW = NC * NS
L = 16

HALF = N_NODES // NC
TRASH = HALF
ACC_ROWS = 50048

OCH = 128
NZCH = ACC_ROWS // OCH
N_FULL_CHUNKS = HALF // OCH
REM_ROWS = HALF - N_FULL_CHUNKS * OCH

CHUNK = 128
GK = 16
GE = GK * CHUNK
CAPC = 12800
EPC = CAPC * CHUNK

SCN = N_EDGES // NW
SGRP = 2000
NSGRP = SCN // SGRP
FIFO = 4096


def _zero_vec():
  return jnp.zeros((L,), jnp.float32)


def _count_body(rows_h, counts_h, rbuf, cbuf):
  c = lax.axis_index("c")
  s = lax.axis_index("s")
  wid = c * NS + s

  def group(g, cnt):
    pltpu.sync_copy(rows_h.at[pl.ds(wid * SCN + g * SGRP, SGRP)], rbuf)

    def step(i, cnt2):
      rr = rbuf[pl.ds(i * L, L)]
      return cnt2 + jnp.where(rr < HALF, 1, 0).astype(jnp.int32)
    return lax.fori_loop(0, SGRP // L, step, cnt)

  cnt = lax.fori_loop(0, NSGRP, group, jnp.zeros((L,), jnp.int32))
  total = cnt[0]
  for _i in range(1, L):
    total = total + cnt[_i]
  lanes = lax.iota(jnp.int32, L)
  cbuf[pl.ds(0, L)] = jnp.where(lanes == 0, total, 0)
  pltpu.sync_copy(cbuf, counts_h.at[wid])


def _route_body(rows_h, cols_h, vals_h, counts_h,
                cols2, rows2, vals2, summary,
                rbuf, cbuf, vbuf, cntb, tb,
                fc0, fr0, fv0, fc1, fr1, fv1):
  c = lax.axis_index("c")
  s = lax.axis_index("s")
  wid = c * NS + s
  lanes = lax.iota(jnp.int32, L)

  pltpu.sync_copy(counts_h, cntb)
  base0 = jnp.int32(0)
  base1 = jnp.int32(0)
  ncc0 = jnp.int32(0)
  ncc1 = jnp.int32(0)
  for t in range(NW):
    cv = cntb[t, pl.ds(0, L)]
    c0 = cv[0]
    p0 = (c0 + (CHUNK - 1)) // CHUNK
    p1 = ((SCN - c0) + (CHUNK - 1)) // CHUNK
    is_before = jnp.int32(t) < wid
    base0 = base0 + jnp.where(is_before, p0, 0)
    base1 = base1 + jnp.where(is_before, p1, 0)
    ncc0 = ncc0 + p0
    ncc1 = ncc1 + p1

  fifos = ((fc0, fr0, fv0), (fc1, fr1, fv1))

  def flush(core, fo, dst_edge_off, n):
    fc, fr, fv = fifos[core]
    fo = pl.multiple_of(jnp.int32(fo), CHUNK)
    base = pl.multiple_of(core * EPC + dst_edge_off, CHUNK)
    pltpu.sync_copy(fc.at[pl.ds(fo, n)], cols2.at[pl.ds(base, n)])
    pltpu.sync_copy(fr.at[pl.ds(fo, n)], rows2.at[pl.ds(base, n)])
    pltpu.sync_copy(fv.at[pl.ds(fo, n)], vals2.at[pl.ds(base, n)])

  def scan_group(g, carry):
    off0, off1, fl0, fl1 = carry
    goff = wid * SCN + g * SGRP
    pltpu.sync_copy(rows_h.at[pl.ds(goff, SGRP)], rbuf)
    pltpu.sync_copy(cols_h.at[pl.ds(goff, SGRP)], cbuf)
    pltpu.sync_copy(vals_h.at[pl.ds(goff, SGRP)], vbuf)

    def step(i, carry2):
      o0, o1 = carry2
      rr = rbuf[pl.ds(i * L, L)]
      cc = cbuf[pl.ds(i * L, L)]
      vv = vbuf[pl.ds(i * L, L)]
      m0 = rr < HALF
      x = jnp.where(m0, 1, 0).astype(jnp.int32)
      for sh in (1, 2, 4, 8):
        shifted = x[jnp.maximum(lanes - sh, 0)]
        x = x + jnp.where(lanes >= sh, shifted, 0)
      n0 = x[L - 1]
      dv = lanes + 1
      lo = jnp.zeros((L,), jnp.int32)
      for stp in (8, 4, 2, 1):
        cand = lo + stp
        pc = x[cand - 1]
        lo = jnp.where(pc < dv, cand, lo)
      fc0[pl.ds(o0, L)] = cc[lo]
      fr0[pl.ds(o0, L)] = rr[lo]
      fv0[pl.ds(o0, L)] = vv[lo]
      q = dv - x
      lo1 = jnp.zeros((L,), jnp.int32)
      for stp in (8, 4, 2, 1):
        cand = lo1 + stp
        pc = q[cand - 1]
        lo1 = jnp.where(pc < dv, cand, lo1)
      fc1[pl.ds(o1, L)] = cc[lo1]
      fr1[pl.ds(o1, L)] = rr[lo1] - HALF
      fv1[pl.ds(o1, L)] = vv[lo1]
      return (o0 + n0, o1 + (L - n0))

    off0, off1 = lax.fori_loop(0, SGRP // L, step, (off0, off1))

    def do_flush(core, off, fl, base):
      full = off >= GE

      @pl.when(full)
      def _():
        flush(core, 0, base * CHUNK + fl, GE)
        fc, fr, fv = fifos[core]

        def mv(i, carry3):
          fc[pl.ds(i * L, L)] = fc[pl.ds(GE + i * L, L)]
          fr[pl.ds(i * L, L)] = fr[pl.ds(GE + i * L, L)]
          fv[pl.ds(i * L, L)] = fv[pl.ds(GE + i * L, L)]
          return carry3
        lax.fori_loop(0, GE // L, mv, 0)

      off = jnp.where(full, off - GE, off)
      fl = jnp.where(full, fl + GE, fl)
      return off, fl

    off0, fl0 = do_flush(0, off0, fl0, base0)
    off1, fl1 = do_flush(1, off1, fl1, base1)
    return (off0, off1, fl0, fl1)

  off0, off1, fl0, fl1 = lax.fori_loop(
      0, NSGRP, scan_group,
      (jnp.int32(0), jnp.int32(0), jnp.int32(0), jnp.int32(0)))

  def tail(core, off, fl, base):
    fc, fr, fv = fifos[core]
    pstart = off // L

    def padv(i, carry2):
      b = (pstart + i) * L
      idxv = lanes + b
      m = idxv >= off
      cvv = fc[pl.ds(b, L)]
      rvv = fr[pl.ds(b, L)]
      vvv = fv[pl.ds(b, L)]
      fc[pl.ds(b, L)] = jnp.where(m, 0, cvv)
      fr[pl.ds(b, L)] = jnp.where(m, TRASH, rvv)
      fv[pl.ds(b, L)] = jnp.where(m, 0.0, vvv)
      return carry2
    lax.fori_loop(0, (CHUNK // L) + 1, padv, 0)

    rem_ch = (off + (CHUNK - 1)) // CHUNK
    fo = jnp.int32(0)
    for nch in (16, 8, 4, 2, 1):
      cond = (rem_ch & nch) != 0
      n = nch * CHUNK
      fo_now = fo

      @pl.when(cond)
      def _(core=core, fo_now=fo_now, n=n):
        flush(core, fo_now, base * CHUNK + fl + fo_now, n)
      fo = fo + jnp.where(cond, n, 0)

  tail(0, off0, fl0, base0)
  tail(1, off1, fl1, base1)

  def fill_const(ref, val):
    def f(i, carry2):
      ref[pl.ds(i * L, L)] = jnp.full((L,), val, ref.dtype)
      return carry2
    lax.fori_loop(0, CHUNK // L, f, 0)
  fill_const(fc0, 0)
  fill_const(fr0, TRASH)
  fill_const(fv0, 0.0)

  def pad_region(core, ncc):
    pad_end = ((ncc + 255) // 256) * 256

    def padc(j, carry2):
      ch = ncc + wid + j * NW

      @pl.when(ch < pad_end)
      def _():
        base = pl.multiple_of(core * EPC + ch * CHUNK, CHUNK)
        pltpu.sync_copy(fc0.at[pl.ds(0, CHUNK)], cols2.at[pl.ds(base, CHUNK)])
        pltpu.sync_copy(fr0.at[pl.ds(0, CHUNK)], rows2.at[pl.ds(base, CHUNK)])
        pltpu.sync_copy(fv0.at[pl.ds(0, CHUNK)], vals2.at[pl.ds(base, CHUNK)])
      return carry2
    lax.fori_loop(0, 8, padc, 0)

  pad_region(0, ncc0)
  pad_region(1, ncc1)

  @pl.when(wid == 0)
  def _():
    tb[pl.ds(0, L)] = jnp.where(lanes == 0, ncc0,
                                jnp.where(lanes == 1, ncc1, 0))
    pltpu.sync_copy(tb, summary.at[pl.ds(0, L)])


def _layer_body(finalize, ego_h, cols2, rows2, vals2, summary_h, e1_h, out_h,
                acc, sumv, ecol, erowl, evalv, radj,
                gath0, gath1, gath2, gath3,
                sem0, sem1, sem2, sem3, ssem0, ssem1, ssem2, ssem3):
  obuf, b1, b2 = gath0, gath1, gath2
  c = lax.axis_index("c")
  s = lax.axis_index("s")
  base_row = c * HALF

  pltpu.sync_copy(summary_h, sumv)
  sv = sumv[pl.ds(0, L)]
  ncc = jnp.where(c == 0, sv[0], sv[1])
  ngrp = (ncc + 255) // 256

  def zfill(i, carry):
    obuf[i, pl.ds(0, L)] = _zero_vec()
    obuf[i, pl.ds(L, L)] = _zero_vec()
    return carry
  lax.fori_loop(0, OCH, zfill, 0)

  def zcopy(j, carry):
    cid = s + j * NS
    @pl.when(cid < NZCH)
    def _():
      pltpu.sync_copy(obuf, acc.at[pl.ds(cid * OCH, OCH)])
    return carry
  lax.fori_loop(0, (NZCH + NS - 1) // NS, zcopy, 0)
  plsc.subcore_barrier()

  def group_step(g, carry):
    goff = c * EPC + (s * ngrp + g) * GE
    pltpu.sync_copy(cols2.at[pl.ds(goff, GE)], ecol)
    desc0 = pltpu.async_copy(ego_h.at[ecol.at[pl.ds(0, CHUNK)]], gath0, sem0)
    desc1 = pltpu.async_copy(ego_h.at[ecol.at[pl.ds(CHUNK, CHUNK)]], gath1,
                             sem1)
    gdescs = [desc0, desc1]
    pltpu.sync_copy(rows2.at[pl.ds(goff, GE)], erowl)
    pltpu.sync_copy(vals2.at[pl.ds(goff, GE)], evalv)

    def rcopy_k(k, carry2):
      def rcopy_i(i, carry3):
        radj[k, pl.ds(i * L, L)] = erowl[pl.ds(k * CHUNK + i * L, L)]
        return carry3
      lax.fori_loop(0, CHUNK // L, rcopy_i, 0)
      return carry2
    lax.fori_loop(0, GK, rcopy_k, 0)

    bufs = (gath0, gath1, gath2, gath3)
    gsems = (sem0, sem1, sem2, sem3)
    ssems = (ssem0, ssem1, ssem2, ssem3)
    sdescs = [None] * GK
    for k in range(GK):
      gbuf = bufs[k % 4]
      gdescs[k].wait()
      if k + 2 < GK:
        if k >= 2:
          sdescs[k - 2].wait()
        gdescs.append(pltpu.async_copy(
            ego_h.at[ecol.at[pl.ds((k + 2) * CHUNK, CHUNK)]],
            bufs[(k + 2) % 4], gsems[(k + 2) % 4]))

      def scale_g(i, carry2):
        vg = evalv[pl.ds(k * CHUNK + i * L, L)]
        for lane in range(L):
          e = i * L + lane
          b = jnp.full((L,), vg[lane], jnp.float32)
          gbuf[e, pl.ds(0, L)] = gbuf[e, pl.ds(0, L)] * b
          gbuf[e, pl.ds(L, L)] = gbuf[e, pl.ds(L, L)] * b
        return carry2
      lax.fori_loop(0, CHUNK // L, scale_g, 0, unroll=4)

      sdescs[k] = pltpu.async_copy(
          gbuf, acc.at[radj.at[k]], ssems[k % 4], add=True)
    for k in range(GK - 4, GK):
      sdescs[k].wait()
    return carry
  lax.fori_loop(0, ngrp, group_step, 0)
  plsc.subcore_barrier()

  def emit_chunk(r0, n):
    pltpu.sync_copy(acc.at[pl.ds(r0, n)], obuf.at[pl.ds(0, n)])
    if finalize:
      pltpu.sync_copy(e1_h.at[pl.ds(base_row + r0, n)], b1.at[pl.ds(0, n)])
      pltpu.sync_copy(ego_h.at[pl.ds(base_row + r0, n)], b2.at[pl.ds(0, n)])

      def mean_row(i, carry2):
        third = jnp.full((L,), 1.0 / 3.0, jnp.float32)
        lo = (obuf[i, pl.ds(0, L)] + b1[i, pl.ds(0, L)] + b2[i, pl.ds(0, L)])
        hi = (obuf[i, pl.ds(L, L)] + b1[i, pl.ds(L, L)] + b2[i, pl.ds(L, L)])
        obuf[i, pl.ds(0, L)] = lo * third
        obuf[i, pl.ds(L, L)] = hi * third
        return carry2
      lax.fori_loop(0, n, mean_row, 0)
    pltpu.sync_copy(obuf.at[pl.ds(0, n)], out_h.at[pl.ds(base_row + r0, n)])

  def cout(j, carry):
    cid = s + j * NS
    @pl.when(cid < N_FULL_CHUNKS)
    def _():
      emit_chunk(cid * OCH, OCH)
    return carry
  lax.fori_loop(0, (N_FULL_CHUNKS + NS - 1) // NS, cout, 0)

  @pl.when(s == NS - 1)
  def _():
    emit_chunk(N_FULL_CHUNKS * OCH, REM_ROWS)


def _sc_mesh():
  return plsc.VectorSubcoreMesh(core_axis_name="c", subcore_axis_name="s")


def _make_count():
  return pl.kernel(
      _count_body,
      out_type=jax.ShapeDtypeStruct((NW, L), jnp.int32),
      mesh=_sc_mesh(),
      scratch_types=[
          pltpu.VMEM((SGRP,), jnp.int32),
          pltpu.VMEM((L,), jnp.int32),
      ],
      compiler_params=pltpu.CompilerParams(use_tc_tiling_on_sc=False),
      name="lightgcn_count",
  )


def _make_route():
  return pl.kernel(
      _route_body,
      out_type=(
          jax.ShapeDtypeStruct((NC * EPC,), jnp.int32),
          jax.ShapeDtypeStruct((NC * EPC,), jnp.int32),
          jax.ShapeDtypeStruct((NC * EPC,), jnp.float32),
          jax.ShapeDtypeStruct((L,), jnp.int32),
      ),
      mesh=_sc_mesh(),
      scratch_types=[
          pltpu.VMEM((SGRP,), jnp.int32),
          pltpu.VMEM((SGRP,), jnp.int32),
          pltpu.VMEM((SGRP,), jnp.float32),
          pltpu.VMEM((NW, L), jnp.int32),
          pltpu.VMEM((L,), jnp.int32),
          pltpu.VMEM((FIFO,), jnp.int32),
          pltpu.VMEM((FIFO,), jnp.int32),
          pltpu.VMEM((FIFO,), jnp.float32),
          pltpu.VMEM((FIFO,), jnp.int32),
          pltpu.VMEM((FIFO,), jnp.int32),
          pltpu.VMEM((FIFO,), jnp.float32),
      ],
      compiler_params=pltpu.CompilerParams(use_tc_tiling_on_sc=False),
      name="lightgcn_route",
  )


def _make_layer(finalize):
  return pl.kernel(
      functools.partial(_layer_body, finalize),
      out_type=jax.ShapeDtypeStruct((N_NODES, D), jnp.float32),
      mesh=_sc_mesh(),
      scratch_types=[
          pltpu.VMEM_SHARED((ACC_ROWS, D), jnp.float32),
          pltpu.VMEM((L,), jnp.int32),
          pltpu.VMEM((GE,), jnp.int32),
          pltpu.VMEM((GE,), jnp.int32),
          pltpu.VMEM((GE,), jnp.float32),
          pltpu.VMEM((GK, CHUNK), jnp.int32),
          pltpu.VMEM((CHUNK, D), jnp.float32),
          pltpu.VMEM((CHUNK, D), jnp.float32),
          pltpu.VMEM((CHUNK, D), jnp.float32),
          pltpu.VMEM((CHUNK, D), jnp.float32),
          pltpu.SemaphoreType.DMA,
          pltpu.SemaphoreType.DMA,
          pltpu.SemaphoreType.DMA,
          pltpu.SemaphoreType.DMA,
          pltpu.SemaphoreType.DMA,
          pltpu.SemaphoreType.DMA,
          pltpu.SemaphoreType.DMA,
          pltpu.SemaphoreType.DMA,
      ],
      compiler_params=pltpu.CompilerParams(use_tc_tiling_on_sc=False),
      name="lightgcn_layer_final" if finalize else "lightgcn_layer",
  )


def kernel(user_emb, item_emb, adj_indices, adj_values):
  ego0 = jnp.concatenate([user_emb, item_emb], axis=0)
  rows = adj_indices[0].astype(jnp.int32)
  cols = adj_indices[1].astype(jnp.int32)
  vals = adj_values.astype(jnp.float32)

  counts = _make_count()(rows)
  cols2, rows2, vals2, summary = _make_route()(rows, cols, vals, counts)

  layer = _make_layer(False)
  layer_final = _make_layer(True)

  dummy = jnp.zeros((8, D), jnp.float32)
  e1 = layer(ego0, cols2, rows2, vals2, summary, dummy)
  e2 = layer(e1, cols2, rows2, vals2, summary, dummy)
  out = layer_final(e2, cols2, rows2, vals2, summary, e1)
  return (out[:USER_NUM], out[USER_NUM:])
